# Initial kernel scaffold; baseline (speedup 1.0000x reference)
import jax
import jax.numpy as jnp
from jax.experimental import pallas as pl


def kernel(z, t, conditioning, mask, params):
    raise NotImplementedError("write your pallas kernel here")


# speedup vs baseline: 5.6022x; 5.6022x over previous
import functools

import jax
import jax.numpy as jnp
import numpy as np
from jax import lax
from jax.experimental import pallas as pl
from jax.experimental.pallas import tpu as pltpu
from jax.experimental.pallas import tpu_sc as plsc

KNN_K = 20
NPOS = 3
LAT = 128
NB = 2
NN = 2000
NNP = 2048
EPG = NNP * KNN_K
ETOT = NB * EPG
GTOT = ETOT + NB * NNP
NW = 32
CH = 128
ACC_R = 4096
DUMP = 4000
MP_S = 4
NNB = 400
EBLK = 64


def _sc_gather(table, idx):
    nch = GTOT // (NW * CH)
    mesh = plsc.VectorSubcoreMesh(core_axis_name="c", subcore_axis_name="s")

    @functools.partial(
        pl.kernel,
        out_type=jax.ShapeDtypeStruct((GTOT, LAT), jnp.float32),
        mesh=mesh,
        scratch_types=[
            pltpu.VMEM((CH,), jnp.int32),
            pltpu.VMEM((CH, LAT), jnp.float32),
            pltpu.SemaphoreType.DMA,
        ],
    )
    def gk(table_hbm, idx_hbm, out_hbm, idx_v, rows_v, sem):
        wid = lax.axis_index("s") * 2 + lax.axis_index("c")
        base = wid * (nch * CH)

        def body(j, carry):
            off = base + j * CH
            pltpu.sync_copy(idx_hbm.at[pl.ds(off, CH)], idx_v)
            pltpu.async_copy(table_hbm.at[idx_v], rows_v, sem).wait()
            pltpu.sync_copy(rows_v, out_hbm.at[pl.ds(off, CH)])
            return carry

        lax.fori_loop(0, nch, body, 0)

    return gk(table, idx)


def _sc_scatter_add(msgs, seg, zeros):
    nch = ETOT // (NW * CH)
    mesh = plsc.VectorSubcoreMesh(core_axis_name="c", subcore_axis_name="s")

    @functools.partial(
        pl.kernel,
        out_type=jax.ShapeDtypeStruct((2, ACC_R, LAT), jnp.float32),
        mesh=mesh,
        scratch_types=[
            pltpu.VMEM((CH,), jnp.int32),
            pltpu.VMEM((CH, LAT), jnp.float32),
            pltpu.VMEM_SHARED((ACC_R, LAT), jnp.float32),
            pltpu.SemaphoreType.DMA,
        ],
    )
    def sk(msg_hbm, seg_hbm, zero_hbm, out_hbm, idx_v, rows_v, acc, sem):
        c = lax.axis_index("c")
        s = lax.axis_index("s")
        wid = s * 2 + c
        pltpu.sync_copy(zero_hbm.at[pl.ds(0, 256)], acc.at[pl.ds(s * 256, 256)])
        plsc.subcore_barrier()
        base = wid * (nch * CH)

        def body(j, carry):
            off = base + j * CH
            pltpu.sync_copy(seg_hbm.at[pl.ds(off, CH)], idx_v)
            pltpu.sync_copy(msg_hbm.at[pl.ds(off, CH)], rows_v)
            pltpu.sync_copy(rows_v, acc.at[idx_v], add=True)
            return carry

        lax.fori_loop(0, nch, body, 0)
        plsc.subcore_barrier()
        pltpu.sync_copy(acc.at[pl.ds(s * 256, 128)], rows_v)
        pltpu.sync_copy(rows_v, out_hbm.at[c, pl.ds(s * 256, 128)])
        pltpu.sync_copy(acc.at[pl.ds(s * 256 + 128, 128)], rows_v)
        pltpu.sync_copy(rows_v, out_hbm.at[c, pl.ds(s * 256 + 128, 128)])

    return sk(msgs, seg, zeros)


def _knn_body(rows_ref, cols_ref, tgt_ref):
    rp = rows_ref[0]
    cp = cols_ref[0]
    dm = jnp.zeros((NNB, NN), jnp.float32)
    for d in range(NPOS):
        diff = rp[:, d:d + 1] - cp[d:d + 1, :]
        dm = dm + diff * diff
    big = jnp.float32(1e10)
    dm = jnp.where(cp[3:4, :] > 0, dm, big)
    dm = jnp.where(rp[:, 3:4] > 0, dm, big)
    iota = lax.broadcasted_iota(jnp.int32, (NNB, NN), 1)
    inf = jnp.float32(np.inf)
    for t in range(KNN_K):
        vmin = jnp.min(dm, axis=1, keepdims=True)
        cand = jnp.where(dm == vmin, iota, jnp.int32(1 << 30))
        imin = jnp.min(cand, axis=1, keepdims=True)
        tgt_ref[0, :, t:t + 1] = imin
        dm = jnp.where(iota == imin, inf, dm)


def _knn(rows_p, cols_p):
    nb = NN // NNB
    return pl.pallas_call(
        _knn_body,
        grid=(NB, nb),
        in_specs=[
            pl.BlockSpec((1, NNB, 8), lambda g, r: (g, r, 0)),
            pl.BlockSpec((1, 8, NN), lambda g, r: (g, 0, 0)),
        ],
        out_specs=pl.BlockSpec((1, NNB, KNN_K), lambda g, r: (g, r, 0)),
        out_shape=jax.ShapeDtypeStruct((NB, NN, KNN_K), jnp.int32),
    )(rows_p, cols_p)


def _prep_body(cv_ref, w1, b1, w2, b2, w3, b3, wg, bg, we, be, wn, bn,
               ce_ref, cn_ref):
    li = lax.broadcasted_iota(jnp.int32, (8, LAT), 1)
    x = cv_ref[...]
    x = jnp.where(li < 16, jnp.sin(x), jnp.where(li < 32, jnp.cos(x), x))
    x = jnp.where(li < 34, x, jnp.float32(0.0))
    x = jax.nn.gelu(jnp.dot(x, w1[...], preferred_element_type=jnp.float32) + b1[0:1])
    x = jax.nn.gelu(jnp.dot(x, w2[...], preferred_element_type=jnp.float32) + b2[0:1])
    x = jnp.dot(x, w3[...], preferred_element_type=jnp.float32) + b3[0:1]
    g = jnp.dot(x, wg[...], preferred_element_type=jnp.float32) + bg[0:1]
    for s in range(MP_S):
        ce_ref[s] = jnp.dot(g, we[s], preferred_element_type=jnp.float32) + be[s, 0:1]
        cn_ref[s] = jnp.dot(g, wn[s], preferred_element_type=jnp.float32) + bn[s, 0:1]


def _prep(cv, w1, b1, w2, b2, w3, b3, wg, bg, we, be, wn, bn):
    spec0 = lambda shp: pl.BlockSpec(shp, lambda: tuple(0 for _ in shp))
    args = [cv, w1, b1, w2, b2, w3, b3, wg, bg, we, be, wn, bn]
    return pl.pallas_call(
        _prep_body,
        grid=(),
        in_specs=[spec0(a.shape) for a in args],
        out_specs=[spec0((MP_S, 8, LAT)), spec0((MP_S, 8, LAT))],
        out_shape=[jax.ShapeDtypeStruct((MP_S, 8, LAT), jnp.float32)] * 2,
    )(*args)


def _embed_body(z_ref, w_ref, b_ref, out_ref):
    out_ref[0] = (jnp.dot(z_ref[0], w_ref[...], preferred_element_type=jnp.float32)
                  + b_ref[0:1])


def _embed(zp, w, b):
    nb = NN // NNB
    return pl.pallas_call(
        _embed_body,
        grid=(NB, nb),
        in_specs=[
            pl.BlockSpec((1, NNB, 8), lambda g, r: (g, r, 0)),
            pl.BlockSpec((8, LAT), lambda g, r: (0, 0)),
            pl.BlockSpec((8, LAT), lambda g, r: (0, 0)),
        ],
        out_specs=pl.BlockSpec((1, NNB, LAT), lambda g, r: (g, r, 0)),
        out_shape=jax.ShapeDtypeStruct((NB, NN, LAT), jnp.float32),
    )(zp, w, b)


def _hab_body(h_ref, w_ref, bias_ref, out_ref):
    out_ref[0] = (jnp.dot(h_ref[0], w_ref[0], preferred_element_type=jnp.float32)
                  + bias_ref[0, 0, 0:1])


def _hab(h, wab, bias):
    return pl.pallas_call(
        _hab_body,
        grid=(NB, 2, 2),
        in_specs=[
            pl.BlockSpec((1, 1000, LAT), lambda g, j, r: (g, r, 0)),
            pl.BlockSpec((1, LAT, LAT), lambda g, j, r: (j, 0, 0)),
            pl.BlockSpec((1, 1, 8, LAT), lambda g, j, r: (g, j, 0, 0)),
        ],
        out_specs=pl.BlockSpec((1, 1000, LAT), lambda g, j, r: (2 * g + j, r, 0)),
        out_shape=jax.ShapeDtypeStruct((2 * NB, NN, LAT), jnp.float32),
    )(h, wab, bias)


def _edge_body(gb_ref, ha_ref, w2, b2, w3, b3, w4, b4, out_ref):
    ha = ha_ref[...]
    ha = jnp.broadcast_to(ha[:, None, :], (EBLK, KNN_K, LAT))
    ha = ha.reshape(EBLK * KNN_K, LAT)
    x = gb_ref[...] + ha
    x = jax.nn.gelu(x)
    x = jnp.dot(x, w2[...], preferred_element_type=jnp.float32) + b2[0:1]
    x = jax.nn.gelu(x)
    x = jnp.dot(x, w3[...], preferred_element_type=jnp.float32) + b3[0:1]
    x = jax.nn.gelu(x)
    out_ref[...] = jnp.dot(x, w4[...], preferred_element_type=jnp.float32) + b4[0:1]


def _edge(gath, w2, b2, w3, b3, w4, b4):
    ne = NNP // EBLK
    eb = EBLK * KNN_K
    wspec = pl.BlockSpec((LAT, LAT), lambda g, e: (0, 0))
    bspec = pl.BlockSpec((8, LAT), lambda g, e: (0, 0))
    return pl.pallas_call(
        _edge_body,
        grid=(NB, ne),
        in_specs=[
            pl.BlockSpec((eb, LAT), lambda g, e: (g * ne + e, 0)),
            pl.BlockSpec((EBLK, LAT), lambda g, e: (ETOT // EBLK + g * ne + e, 0)),
            wspec, bspec, wspec, bspec, wspec, bspec,
        ],
        out_specs=pl.BlockSpec((eb, LAT), lambda g, e: (g * ne + e, 0)),
        out_shape=jax.ShapeDtypeStruct((ETOT, LAT), jnp.float32),
    )(gath, gath, w2, b2, w3, b3, w4, b4)


def _node_body(h_ref, p0_ref, p1_ref, w1a, w1b, cn_ref, w2, b2, w3, b3, w4, b4,
               out_ref):
    h = h_ref[0]
    agg = p0_ref[0] + p1_ref[0]
    x = (jnp.dot(h, w1a[...], preferred_element_type=jnp.float32)
         + jnp.dot(agg, w1b[...], preferred_element_type=jnp.float32)
         + cn_ref[0, 0:1])
    x = jax.nn.gelu(x)
    x = jnp.dot(x, w2[...], preferred_element_type=jnp.float32) + b2[0:1]
    x = jax.nn.gelu(x)
    x = jnp.dot(x, w3[...], preferred_element_type=jnp.float32) + b3[0:1]
    x = jax.nn.gelu(x)
    x = jnp.dot(x, w4[...], preferred_element_type=jnp.float32) + b4[0:1]
    out_ref[0] = h + x


def _node(h, parts, w1a, w1b, cn, w2, b2, w3, b3, w4, b4):
    nb = NN // NNB
    wspec = pl.BlockSpec((LAT, LAT), lambda g, r: (0, 0))
    bspec = pl.BlockSpec((8, LAT), lambda g, r: (0, 0))
    return pl.pallas_call(
        _node_body,
        grid=(NB, nb),
        in_specs=[
            pl.BlockSpec((1, NNB, LAT), lambda g, r: (g, r, 0)),
            pl.BlockSpec((1, NNB, LAT), lambda g, r: (0, g * nb + r, 0)),
            pl.BlockSpec((1, NNB, LAT), lambda g, r: (1, g * nb + r, 0)),
            wspec, wspec,
            pl.BlockSpec((1, 8, LAT), lambda g, r: (g, 0, 0)),
            wspec, bspec, wspec, bspec, wspec, bspec,
        ],
        out_specs=pl.BlockSpec((1, NNB, LAT), lambda g, r: (g, r, 0)),
        out_shape=jax.ShapeDtypeStruct((NB, NN, LAT), jnp.float32),
    )(h, parts, parts, w1a, w1b, cn, w2, b2, w3, b3, w4, b4)


def _dec_body(h_ref, zp_ref, w1, b1, w2, b2, w3, b3, w4, b4, out_ref):
    x = h_ref[0]
    x = jax.nn.gelu(jnp.dot(x, w1[...], preferred_element_type=jnp.float32) + b1[0:1])
    x = jax.nn.gelu(jnp.dot(x, w2[...], preferred_element_type=jnp.float32) + b2[0:1])
    x = jax.nn.gelu(jnp.dot(x, w3[...], preferred_element_type=jnp.float32) + b3[0:1])
    x = jnp.dot(x, w4[...], preferred_element_type=jnp.float32) + b4[0:1]
    out_ref[0] = zp_ref[0] - x[:, 0:8]


def _decode(h, zp, w1, b1, w2, b2, w3, b3, w4, b4):
    nb = NN // NNB
    wspec = pl.BlockSpec((LAT, LAT), lambda g, r: (0, 0))
    bspec = pl.BlockSpec((8, LAT), lambda g, r: (0, 0))
    return pl.pallas_call(
        _dec_body,
        grid=(NB, nb),
        in_specs=[
            pl.BlockSpec((1, NNB, LAT), lambda g, r: (g, r, 0)),
            pl.BlockSpec((1, NNB, 8), lambda g, r: (g, r, 0)),
            wspec, bspec, wspec, bspec, wspec, bspec, wspec, bspec,
        ],
        out_specs=pl.BlockSpec((1, NNB, 8), lambda g, r: (g, r, 0)),
        out_shape=jax.ShapeDtypeStruct((NB, NN, 8), jnp.float32),
    )(h, zp, w1, b1, w2, b2, w3, b3, w4, b4)


def _pad2(w, r, c):
    return jnp.pad(w, ((0, r - w.shape[0]), (0, c - w.shape[1])))


def _brow(b, c):
    return jnp.broadcast_to(jnp.pad(b, (0, c - b.shape[0]))[None, :], (8, c))


def kernel(z, t, conditioning, mask, params):
    f32 = jnp.float32
    z = z.astype(f32)

    half = 16
    freq = jnp.exp(jnp.arange(half, dtype=f32) * (-np.log(10000.0) / (half - 1)))
    phase = (t.astype(f32) * 1000.0)[:, None] * freq[None, :]
    cv = jnp.zeros((8, LAT), f32)
    cv = cv.at[0:NB, 0:half].set(phase)
    cv = cv.at[0:NB, half:2 * half].set(phase)
    cv = cv.at[0:NB, 2 * half:2 * half + 2].set(conditioning.astype(f32))

    (w1c, b1c), (w2c, b2c), (w3c, b3c) = params['cond_mlp']
    wg, bg = params['global_embed']
    we = jnp.stack([_pad2(params['edge_mlps'][s][0][0][256:384], LAT, LAT)
                    for s in range(MP_S)])
    be = jnp.stack([_brow(params['edge_mlps'][s][0][1], LAT) for s in range(MP_S)])
    wn = jnp.stack([_pad2(params['node_mlps'][s][0][0][256:384], LAT, LAT)
                    for s in range(MP_S)])
    bn = jnp.stack([_brow(params['node_mlps'][s][0][1], LAT) for s in range(MP_S)])
    cedge, cnode = _prep(
        cv, _pad2(w1c, LAT, 256), _brow(b1c, 256), _pad2(w2c, 256, 256),
        _brow(b2c, 256), _pad2(w3c, 256, LAT), _brow(b3c, LAT),
        _pad2(wg, LAT, LAT), _brow(bg, LAT), we, be, wn, bn)

    maskf = mask.astype(f32)
    pos = z[:, :, :NPOS]
    rows_p = jnp.concatenate(
        [pos, maskf[:, :, None], jnp.zeros((NB, NN, 4), f32)], axis=2)
    cols_p = jnp.swapaxes(rows_p, 1, 2)
    tgt = _knn(rows_p, cols_p)
    src0 = tgt[:, :, 0]

    goff = (jnp.arange(NB, dtype=jnp.int32) * 2 * NN)[:, None, None]
    gidx_e = jnp.pad(tgt + (goff + NN), ((0, 0), (0, NNP - NN), (0, 0)))
    gidx_s = jnp.pad(src0 + goff[:, :, 0], ((0, 0), (0, NNP - NN)))
    gidx = jnp.concatenate([gidx_e.reshape(-1), gidx_s.reshape(-1)])
    seg = jnp.pad(tgt + (jnp.arange(NB, dtype=jnp.int32) * NN)[:, None, None],
                  ((0, 0), (0, NNP - NN), (0, 0)), constant_values=DUMP)
    seg = seg.reshape(-1)
    zeros256 = jnp.zeros((256, LAT), f32)

    wemb, bemb = params['node_embed']
    zp = jnp.concatenate([z, jnp.zeros((NB, NN, 1), f32)], axis=2)
    h = _embed(zp, _pad2(wemb, 8, LAT), _brow(bemb, LAT))

    for s in range(MP_S):
        ew = params['edge_mlps'][s]
        nw_ = params['node_mlps'][s]
        wab = jnp.stack([ew[0][0][0:128], ew[0][0][128:256]])
        biasE = jnp.concatenate(
            [jnp.broadcast_to(cedge[s][0:NB][:, None, None, :], (NB, 1, 8, LAT)),
             jnp.zeros((NB, 1, 8, LAT), f32)], axis=1)
        tab = _hab(h, wab, biasE).reshape(2 * NB * NN, LAT)
        gath = _sc_gather(tab, gidx)
        msgs = _edge(gath, ew[1][0], _brow(ew[1][1], LAT), ew[2][0],
                     _brow(ew[2][1], LAT), ew[3][0], _brow(ew[3][1], LAT))
        parts = _sc_scatter_add(msgs, seg, zeros256)
        cn = jnp.broadcast_to(cnode[s][0:NB][:, None, :], (NB, 8, LAT))
        h = _node(h, parts, nw_[0][0][0:128], nw_[0][0][128:256], cn,
                  nw_[1][0], _brow(nw_[1][1], LAT), nw_[2][0], _brow(nw_[2][1], LAT),
                  nw_[3][0], _brow(nw_[3][1], LAT))

    dw = params['decoder']
    eps8 = _decode(h, zp, dw[0][0], _brow(dw[0][1], LAT), dw[1][0],
                   _brow(dw[1][1], LAT), dw[2][0], _brow(dw[2][1], LAT),
                   _pad2(dw[3][0], LAT, LAT), _brow(dw[3][1], LAT))
    return eps8[:, :, :7]


# speedup vs baseline: 6.5313x; 1.1658x over previous
import functools

import jax
import jax.numpy as jnp
import numpy as np
from jax import lax
from jax.experimental import pallas as pl
from jax.experimental.pallas import tpu as pltpu
from jax.experimental.pallas import tpu_sc as plsc

KNN_K = 20
NPOS = 3
LAT = 128
NB = 2
NN = 2000
NNP = 2048
EPG = NNP * KNN_K
ETOT = NB * EPG
GTOT = ETOT + NB * NNP
NW = 32
CH = 128
ACC_R = 4096
DUMP = 4000
MP_S = 4
NNB = 400
EBLK = 64


def _sc_gather(table, idx2):
    nch = GTOT // (NW * CH)
    nset = 3
    mesh = plsc.VectorSubcoreMesh(core_axis_name="c", subcore_axis_name="s")

    @functools.partial(
        pl.kernel,
        out_type=jax.ShapeDtypeStruct((GTOT, LAT), jnp.float32),
        mesh=mesh,
        scratch_types=[
            pltpu.VMEM((24, CH), jnp.int32),
            pltpu.VMEM((2 * nset, CH, LAT), jnp.float32),
            pltpu.SemaphoreType.DMA,
            pltpu.SemaphoreType.DMA,
            pltpu.SemaphoreType.DMA,
        ],
    )
    def gk(table_hbm, idx_hbm, out_hbm, idx_v, bufs, gsem0, gsem1, osem):
        wid = lax.axis_index("s") * 2 + lax.axis_index("c")
        base = wid * (nch * CH)
        gsems = (gsem0, gsem1)
        pltpu.sync_copy(idx_hbm.at[pl.ds(wid * 24, 24)], idx_v)
        ngrp = (nch + nset - 1) // nset

        def gcopy(g, setb, b):
            j = g * nset + b
            return pltpu.make_async_copy(
                table_hbm.at[idx_v.at[j]], bufs.at[setb * nset + b], gsems[setb])

        def ocopy(g, setb, b):
            j = g * nset + b
            return pltpu.make_async_copy(
                bufs.at[setb * nset + b], out_hbm.at[pl.ds(base + j * CH, CH)], osem)

        def fire(g, setb):
            for b in range(nset):
                if g * nset + b < nch:
                    gcopy(g, setb, b).start()

        def drain_writes(g, setb):
            for b in range(nset):
                if g * nset + b < nch:
                    gcopy(g, setb, b).wait()
            for b in range(nset):
                if g * nset + b < nch:
                    ocopy(g, setb, b).start()
            for b in range(nset):
                if g * nset + b < nch:
                    ocopy(g, setb, b).wait()

        fire(0, 0)
        for g in range(ngrp):
            if g + 1 < ngrp:
                fire(g + 1, (g + 1) % 2)
            drain_writes(g, g % 2)

    return gk(table, idx2)


def _sc_scatter_add(msgs, seg, zeros):
    nch = ETOT // (NW * CH)
    nbuf = 3
    mesh = plsc.VectorSubcoreMesh(core_axis_name="c", subcore_axis_name="s")

    @functools.partial(
        pl.kernel,
        out_type=jax.ShapeDtypeStruct((2, ACC_R, LAT), jnp.float32),
        mesh=mesh,
        scratch_types=[
            pltpu.VMEM((24, CH), jnp.int32),
            pltpu.VMEM((nbuf, CH, LAT), jnp.float32),
            pltpu.VMEM_SHARED((ACC_R, LAT), jnp.float32),
            pltpu.SemaphoreType.DMA,
            pltpu.SemaphoreType.DMA,
            pltpu.SemaphoreType.DMA,
            pltpu.SemaphoreType.DMA,
        ],
    )
    def sk(msg_hbm, seg_hbm, zero_hbm, out_hbm, seg_v, bufs, acc,
           lsem0, lsem1, lsem2, osem):
        c = lax.axis_index("c")
        s = lax.axis_index("s")
        wid = s * 2 + c
        lsems = (lsem0, lsem1, lsem2)
        base = wid * (nch * CH)
        zcopy = pltpu.make_async_copy(
            zero_hbm.at[pl.ds(0, 256)], acc.at[pl.ds(s * 256, 256)], osem)
        zcopy.start()
        pltpu.sync_copy(seg_hbm.at[pl.ds(wid * 24, 24)], seg_v)

        def lcopy(j, b):
            return pltpu.make_async_copy(
                msg_hbm.at[pl.ds(base + j * CH, CH)], bufs.at[b], lsems[b])

        for j in range(nbuf):
            lcopy(j, j).start()
        zcopy.wait()
        plsc.subcore_barrier()
        for j in range(nch):
            b = j % nbuf
            lcopy(j, b).wait()
            pltpu.sync_copy(bufs.at[b], acc.at[seg_v.at[j]], add=True)
            if j + nbuf < nch:
                lcopy(j + nbuf, b).start()
        plsc.subcore_barrier()
        pltpu.sync_copy(acc.at[pl.ds(s * 256, 128)], bufs.at[0])
        pltpu.make_async_copy(
            bufs.at[0], out_hbm.at[c, pl.ds(s * 256, 128)], osem).start()
        pltpu.sync_copy(acc.at[pl.ds(s * 256 + 128, 128)], bufs.at[1])
        pltpu.make_async_copy(
            bufs.at[1], out_hbm.at[c, pl.ds(s * 256 + 128, 128)], osem).start()
        pltpu.make_async_copy(
            bufs.at[0], out_hbm.at[c, pl.ds(s * 256, 128)], osem).wait()
        pltpu.make_async_copy(
            bufs.at[1], out_hbm.at[c, pl.ds(s * 256 + 128, 128)], osem).wait()

    return sk(msgs, seg, zeros)


def _knn_body(rows_ref, cols_ref, tgt_ref):
    rp = rows_ref[0]
    cp = cols_ref[0]
    dm = jnp.zeros((NNB, NN), jnp.float32)
    for d in range(NPOS):
        diff = rp[:, d:d + 1] - cp[d:d + 1, :]
        dm = dm + diff * diff
    big = jnp.float32(1e10)
    dm = jnp.where(cp[3:4, :] > 0, dm, big)
    dm = jnp.where(rp[:, 3:4] > 0, dm, big)
    iota = lax.broadcasted_iota(jnp.int32, (NNB, NN), 1)
    inf = jnp.float32(np.inf)
    for t in range(KNN_K):
        vmin = jnp.min(dm, axis=1, keepdims=True)
        cand = jnp.where(dm == vmin, iota, jnp.int32(1 << 30))
        imin = jnp.min(cand, axis=1, keepdims=True)
        tgt_ref[0, :, t:t + 1] = imin
        dm = jnp.where(iota == imin, inf, dm)


def _knn(rows_p, cols_p):
    nb = NN // NNB
    return pl.pallas_call(
        _knn_body,
        grid=(NB, nb),
        in_specs=[
            pl.BlockSpec((1, NNB, 8), lambda g, r: (g, r, 0)),
            pl.BlockSpec((1, 8, NN), lambda g, r: (g, 0, 0)),
        ],
        out_specs=pl.BlockSpec((1, NNB, KNN_K), lambda g, r: (g, r, 0)),
        out_shape=jax.ShapeDtypeStruct((NB, NN, KNN_K), jnp.int32),
    )(rows_p, cols_p)


def _prep_body(cv_ref, w1, b1, w2, b2, w3, b3, wg, bg, we, be, wn, bn,
               ce_ref, cn_ref):
    li = lax.broadcasted_iota(jnp.int32, (8, LAT), 1)
    x = cv_ref[...]
    x = jnp.where(li < 16, jnp.sin(x), jnp.where(li < 32, jnp.cos(x), x))
    x = jnp.where(li < 34, x, jnp.float32(0.0))
    x = jax.nn.gelu(jnp.dot(x, w1[...], preferred_element_type=jnp.float32) + b1[0:1])
    x = jax.nn.gelu(jnp.dot(x, w2[...], preferred_element_type=jnp.float32) + b2[0:1])
    x = jnp.dot(x, w3[...], preferred_element_type=jnp.float32) + b3[0:1]
    g = jnp.dot(x, wg[...], preferred_element_type=jnp.float32) + bg[0:1]
    for s in range(MP_S):
        ce_ref[s] = jnp.dot(g, we[s], preferred_element_type=jnp.float32) + be[s, 0:1]
        cn_ref[s] = jnp.dot(g, wn[s], preferred_element_type=jnp.float32) + bn[s, 0:1]


def _prep(cv, w1, b1, w2, b2, w3, b3, wg, bg, we, be, wn, bn):
    spec0 = lambda shp: pl.BlockSpec(shp, lambda: tuple(0 for _ in shp))
    args = [cv, w1, b1, w2, b2, w3, b3, wg, bg, we, be, wn, bn]
    return pl.pallas_call(
        _prep_body,
        grid=(),
        in_specs=[spec0(a.shape) for a in args],
        out_specs=[spec0((MP_S, 8, LAT)), spec0((MP_S, 8, LAT))],
        out_shape=[jax.ShapeDtypeStruct((MP_S, 8, LAT), jnp.float32)] * 2,
    )(*args)


def _embed_body(z_ref, w_ref, b_ref, out_ref):
    out_ref[0] = (jnp.dot(z_ref[0], w_ref[...], preferred_element_type=jnp.float32)
                  + b_ref[0:1])


def _embed(zp, w, b):
    nb = NN // NNB
    return pl.pallas_call(
        _embed_body,
        grid=(NB, nb),
        in_specs=[
            pl.BlockSpec((1, NNB, 8), lambda g, r: (g, r, 0)),
            pl.BlockSpec((8, LAT), lambda g, r: (0, 0)),
            pl.BlockSpec((8, LAT), lambda g, r: (0, 0)),
        ],
        out_specs=pl.BlockSpec((1, NNB, LAT), lambda g, r: (g, r, 0)),
        out_shape=jax.ShapeDtypeStruct((NB, NN, LAT), jnp.float32),
    )(zp, w, b)


def _hab_body(h_ref, w_ref, bias_ref, out_ref):
    out_ref[0] = (jnp.dot(h_ref[0], w_ref[0], preferred_element_type=jnp.float32)
                  + bias_ref[0, 0, 0:1])


def _hab(h, wab, bias):
    return pl.pallas_call(
        _hab_body,
        grid=(NB, 2, 2),
        in_specs=[
            pl.BlockSpec((1, 1000, LAT), lambda g, j, r: (g, r, 0)),
            pl.BlockSpec((1, LAT, LAT), lambda g, j, r: (j, 0, 0)),
            pl.BlockSpec((1, 1, 8, LAT), lambda g, j, r: (g, j, 0, 0)),
        ],
        out_specs=pl.BlockSpec((1, 1000, LAT), lambda g, j, r: (2 * g + j, r, 0)),
        out_shape=jax.ShapeDtypeStruct((2 * NB, NN, LAT), jnp.float32),
    )(h, wab, bias)


def _edge_body(gb_ref, ha_ref, w2, b2, w3, b3, w4, b4, out_ref):
    ha = ha_ref[...]
    ha = jnp.broadcast_to(ha[:, None, :], (EBLK, KNN_K, LAT))
    ha = ha.reshape(EBLK * KNN_K, LAT)
    x = gb_ref[...] + ha
    x = jax.nn.gelu(x)
    x = jnp.dot(x, w2[...], preferred_element_type=jnp.float32) + b2[0:1]
    x = jax.nn.gelu(x)
    x = jnp.dot(x, w3[...], preferred_element_type=jnp.float32) + b3[0:1]
    x = jax.nn.gelu(x)
    out_ref[...] = jnp.dot(x, w4[...], preferred_element_type=jnp.float32) + b4[0:1]


def _edge(gath, w2, b2, w3, b3, w4, b4):
    ne = NNP // EBLK
    eb = EBLK * KNN_K
    wspec = pl.BlockSpec((LAT, LAT), lambda g, e: (0, 0))
    bspec = pl.BlockSpec((8, LAT), lambda g, e: (0, 0))
    return pl.pallas_call(
        _edge_body,
        grid=(NB, ne),
        in_specs=[
            pl.BlockSpec((eb, LAT), lambda g, e: (g * ne + e, 0)),
            pl.BlockSpec((EBLK, LAT), lambda g, e: (ETOT // EBLK + g * ne + e, 0)),
            wspec, bspec, wspec, bspec, wspec, bspec,
        ],
        out_specs=pl.BlockSpec((eb, LAT), lambda g, e: (g * ne + e, 0)),
        out_shape=jax.ShapeDtypeStruct((ETOT, LAT), jnp.float32),
    )(gath, gath, w2, b2, w3, b3, w4, b4)


def _node_body(h_ref, p0_ref, p1_ref, w1a, w1b, cn_ref, w2, b2, w3, b3, w4, b4,
               out_ref):
    h = h_ref[0]
    agg = p0_ref[0] + p1_ref[0]
    x = (jnp.dot(h, w1a[...], preferred_element_type=jnp.float32)
         + jnp.dot(agg, w1b[...], preferred_element_type=jnp.float32)
         + cn_ref[0, 0:1])
    x = jax.nn.gelu(x)
    x = jnp.dot(x, w2[...], preferred_element_type=jnp.float32) + b2[0:1]
    x = jax.nn.gelu(x)
    x = jnp.dot(x, w3[...], preferred_element_type=jnp.float32) + b3[0:1]
    x = jax.nn.gelu(x)
    x = jnp.dot(x, w4[...], preferred_element_type=jnp.float32) + b4[0:1]
    out_ref[0] = h + x


def _node(h, parts, w1a, w1b, cn, w2, b2, w3, b3, w4, b4):
    nb = NN // NNB
    wspec = pl.BlockSpec((LAT, LAT), lambda g, r: (0, 0))
    bspec = pl.BlockSpec((8, LAT), lambda g, r: (0, 0))
    return pl.pallas_call(
        _node_body,
        grid=(NB, nb),
        in_specs=[
            pl.BlockSpec((1, NNB, LAT), lambda g, r: (g, r, 0)),
            pl.BlockSpec((1, NNB, LAT), lambda g, r: (0, g * nb + r, 0)),
            pl.BlockSpec((1, NNB, LAT), lambda g, r: (1, g * nb + r, 0)),
            wspec, wspec,
            pl.BlockSpec((1, 8, LAT), lambda g, r: (g, 0, 0)),
            wspec, bspec, wspec, bspec, wspec, bspec,
        ],
        out_specs=pl.BlockSpec((1, NNB, LAT), lambda g, r: (g, r, 0)),
        out_shape=jax.ShapeDtypeStruct((NB, NN, LAT), jnp.float32),
    )(h, parts, parts, w1a, w1b, cn, w2, b2, w3, b3, w4, b4)


def _dec_body(h_ref, zp_ref, w1, b1, w2, b2, w3, b3, w4, b4, out_ref):
    x = h_ref[0]
    x = jax.nn.gelu(jnp.dot(x, w1[...], preferred_element_type=jnp.float32) + b1[0:1])
    x = jax.nn.gelu(jnp.dot(x, w2[...], preferred_element_type=jnp.float32) + b2[0:1])
    x = jax.nn.gelu(jnp.dot(x, w3[...], preferred_element_type=jnp.float32) + b3[0:1])
    x = jnp.dot(x, w4[...], preferred_element_type=jnp.float32) + b4[0:1]
    out_ref[0] = zp_ref[0] - x[:, 0:8]


def _decode(h, zp, w1, b1, w2, b2, w3, b3, w4, b4):
    nb = NN // NNB
    wspec = pl.BlockSpec((LAT, LAT), lambda g, r: (0, 0))
    bspec = pl.BlockSpec((8, LAT), lambda g, r: (0, 0))
    return pl.pallas_call(
        _dec_body,
        grid=(NB, nb),
        in_specs=[
            pl.BlockSpec((1, NNB, LAT), lambda g, r: (g, r, 0)),
            pl.BlockSpec((1, NNB, 8), lambda g, r: (g, r, 0)),
            wspec, bspec, wspec, bspec, wspec, bspec, wspec, bspec,
        ],
        out_specs=pl.BlockSpec((1, NNB, 8), lambda g, r: (g, r, 0)),
        out_shape=jax.ShapeDtypeStruct((NB, NN, 8), jnp.float32),
    )(h, zp, w1, b1, w2, b2, w3, b3, w4, b4)


def _pad2(w, r, c):
    return jnp.pad(w, ((0, r - w.shape[0]), (0, c - w.shape[1])))


def _brow(b, c):
    return jnp.broadcast_to(jnp.pad(b, (0, c - b.shape[0]))[None, :], (8, c))


def kernel(z, t, conditioning, mask, params):
    f32 = jnp.float32
    z = z.astype(f32)

    half = 16
    freq = jnp.exp(jnp.arange(half, dtype=f32) * (-np.log(10000.0) / (half - 1)))
    phase = (t.astype(f32) * 1000.0)[:, None] * freq[None, :]
    cv = jnp.zeros((8, LAT), f32)
    cv = cv.at[0:NB, 0:half].set(phase)
    cv = cv.at[0:NB, half:2 * half].set(phase)
    cv = cv.at[0:NB, 2 * half:2 * half + 2].set(conditioning.astype(f32))

    (w1c, b1c), (w2c, b2c), (w3c, b3c) = params['cond_mlp']
    wg, bg = params['global_embed']
    we = jnp.stack([_pad2(params['edge_mlps'][s][0][0][256:384], LAT, LAT)
                    for s in range(MP_S)])
    be = jnp.stack([_brow(params['edge_mlps'][s][0][1], LAT) for s in range(MP_S)])
    wn = jnp.stack([_pad2(params['node_mlps'][s][0][0][256:384], LAT, LAT)
                    for s in range(MP_S)])
    bn = jnp.stack([_brow(params['node_mlps'][s][0][1], LAT) for s in range(MP_S)])
    cedge, cnode = _prep(
        cv, _pad2(w1c, LAT, 256), _brow(b1c, 256), _pad2(w2c, 256, 256),
        _brow(b2c, 256), _pad2(w3c, 256, LAT), _brow(b3c, LAT),
        _pad2(wg, LAT, LAT), _brow(bg, LAT), we, be, wn, bn)

    maskf = mask.astype(f32)
    pos = z[:, :, :NPOS]
    rows_p = jnp.concatenate(
        [pos, maskf[:, :, None], jnp.zeros((NB, NN, 4), f32)], axis=2)
    cols_p = jnp.swapaxes(rows_p, 1, 2)
    tgt = _knn(rows_p, cols_p)
    src0 = tgt[:, :, 0]

    goff = (jnp.arange(NB, dtype=jnp.int32) * 2 * NN)[:, None, None]
    gidx_e = jnp.pad(tgt + (goff + NN), ((0, 0), (0, NNP - NN), (0, 0)))
    gidx_s = jnp.pad(src0 + goff[:, :, 0], ((0, 0), (0, NNP - NN)))
    gidx = jnp.concatenate(
        [gidx_e.reshape(-1), gidx_s.reshape(-1)]).reshape(NW, GTOT // (NW * CH), CH)
    gidx = jnp.pad(gidx, ((0, 0), (0, 24 - gidx.shape[1]), (0, 0))).reshape(NW * 24, CH)
    seg = jnp.pad(tgt + (jnp.arange(NB, dtype=jnp.int32) * NN)[:, None, None],
                  ((0, 0), (0, NNP - NN), (0, 0)), constant_values=DUMP)
    seg = seg.reshape(NW, ETOT // (NW * CH), CH)
    seg = jnp.pad(seg, ((0, 0), (0, 24 - seg.shape[1]), (0, 0))).reshape(NW * 24, CH)
    zeros256 = jnp.zeros((256, LAT), f32)

    wemb, bemb = params['node_embed']
    zp = jnp.concatenate([z, jnp.zeros((NB, NN, 1), f32)], axis=2)
    h = _embed(zp, _pad2(wemb, 8, LAT), _brow(bemb, LAT))

    for s in range(MP_S):
        ew = params['edge_mlps'][s]
        nw_ = params['node_mlps'][s]
        wab = jnp.stack([ew[0][0][0:128], ew[0][0][128:256]])
        biasE = jnp.concatenate(
            [jnp.broadcast_to(cedge[s][0:NB][:, None, None, :], (NB, 1, 8, LAT)),
             jnp.zeros((NB, 1, 8, LAT), f32)], axis=1)
        tab = _hab(h, wab, biasE).reshape(2 * NB * NN, LAT)
        gath = _sc_gather(tab, gidx)
        msgs = _edge(gath, ew[1][0], _brow(ew[1][1], LAT), ew[2][0],
                     _brow(ew[2][1], LAT), ew[3][0], _brow(ew[3][1], LAT))
        parts = _sc_scatter_add(msgs, seg, zeros256)
        cn = jnp.broadcast_to(cnode[s][0:NB][:, None, :], (NB, 8, LAT))
        h = _node(h, parts, nw_[0][0][0:128], nw_[0][0][128:256], cn,
                  nw_[1][0], _brow(nw_[1][1], LAT), nw_[2][0], _brow(nw_[2][1], LAT),
                  nw_[3][0], _brow(nw_[3][1], LAT))

    dw = params['decoder']
    eps8 = _decode(h, zp, dw[0][0], _brow(dw[0][1], LAT), dw[1][0],
                   _brow(dw[1][1], LAT), dw[2][0], _brow(dw[2][1], LAT),
                   _pad2(dw[3][0], LAT, LAT), _brow(dw[3][1], LAT))
    return eps8[:, :, :7]


# speedup vs baseline: 9.4472x; 1.4464x over previous
import functools

import jax
import jax.numpy as jnp
import numpy as np
from jax import lax
from jax.experimental import pallas as pl
from jax.experimental.pallas import tpu as pltpu
from jax.experimental.pallas import tpu_sc as plsc

KNN_K = 20
NPOS = 3
LAT = 128
NB = 2
NN = 2000
NNP = 2048
EPG = NNP * KNN_K
ETOT = NB * EPG
GTOT = ETOT + NB * NNP
NW = 32
CH = 128
ACC_R = 4096
DUMP = 4000
MP_S = 4
NNB = 400
EBLK = 64


_SC_CACHE = {}


def _sc_gather(table, idxs):
    nch = 10
    nset = 3
    if "gather" in _SC_CACHE:
        return _SC_CACHE["gather"](table, idxs)
    mesh = plsc.VectorSubcoreMesh(core_axis_name="c", subcore_axis_name="s")

    @functools.partial(
        pl.kernel,
        out_type=(jax.ShapeDtypeStruct((ETOT, LAT), jnp.float32),
                  jax.ShapeDtypeStruct((NB, NW * CH, LAT), jnp.float32)),
        mesh=mesh,
        scratch_types=[
            pltpu.VMEM((16, CH), jnp.int32),
            pltpu.VMEM((2 * nset, CH, LAT), jnp.float32),
            pltpu.VMEM_SHARED((NNP, LAT), jnp.float32),
            pltpu.SemaphoreType.DMA,
            pltpu.SemaphoreType.DMA,
            pltpu.SemaphoreType.DMA,
            pltpu.SemaphoreType.DMA,
        ],
    )
    def gk(table_hbm, idx_hbm, oute_hbm, outs_hbm, idx_v, bufs, tabsh, tsem,
           gsem0, gsem1, osem):
        wid = lax.axis_index("s") * 2 + lax.axis_index("c")
        s = lax.axis_index("s")
        gsems = (gsem0, gsem1)
        ngrp = (nch + nset - 1) // nset

        def stage(g, half):
            tcopy = pltpu.make_async_copy(
                table_hbm.at[g, pl.ds(half * NNP + s * (NNP // 16), NNP // 16)],
                tabsh.at[pl.ds(s * (NNP // 16), NNP // 16)], tsem)
            tcopy.start()
            return tcopy

        for g in range(NB):
            tcopy = stage(g, 1)
            pltpu.sync_copy(idx_hbm.at[g, pl.ds(wid * 16, 16)], idx_v)
            tcopy.wait()
            plsc.subcore_barrier()

            def gcopy(grp, setb, b):
                j = grp * nset + b
                return pltpu.make_async_copy(
                    tabsh.at[idx_v.at[j]], bufs.at[setb * nset + b], gsems[setb])

            def ocopy(grp, setb, b):
                j = grp * nset + b
                return pltpu.make_async_copy(
                    bufs.at[setb * nset + b],
                    oute_hbm.at[pl.ds(g * EPG + wid * 1280 + j * CH, CH)], osem)

            def fire(grp, setb):
                for b in range(nset):
                    if grp * nset + b < nch:
                        gcopy(grp, setb, b).start()

            def drain_writes(grp, setb):
                for b in range(nset):
                    if grp * nset + b < nch:
                        gcopy(grp, setb, b).wait()
                for b in range(nset):
                    if grp * nset + b < nch:
                        ocopy(grp, setb, b).start()
                for b in range(nset):
                    if grp * nset + b < nch:
                        ocopy(grp, setb, b).wait()

            fire(0, 0)
            for grp in range(ngrp):
                if grp + 1 < ngrp:
                    fire(grp + 1, (grp + 1) % 2)
                drain_writes(grp, grp % 2)
            plsc.subcore_barrier()
            stage(g, 0).wait()
            plsc.subcore_barrier()
            srcg = pltpu.make_async_copy(
                tabsh.at[idx_v.at[10]], bufs.at[0], gsem0)
            srcg.start()
            srcg.wait()
            pltpu.sync_copy(bufs.at[0], outs_hbm.at[g, pl.ds(wid * CH, CH)])
            plsc.subcore_barrier()

    _SC_CACHE["gather"] = gk
    return gk(table, idxs)


def _sc_scatter_add(msgs, seg, zeros):
    nch = ETOT // (NW * CH)
    nbuf = 3
    if "scatter" in _SC_CACHE:
        return _SC_CACHE["scatter"](msgs, seg, zeros)
    mesh = plsc.VectorSubcoreMesh(core_axis_name="c", subcore_axis_name="s")

    @functools.partial(
        pl.kernel,
        out_type=jax.ShapeDtypeStruct((2, ACC_R, LAT), jnp.float32),
        mesh=mesh,
        scratch_types=[
            pltpu.VMEM((24, CH), jnp.int32),
            pltpu.VMEM((nbuf, CH, LAT), jnp.float32),
            pltpu.VMEM_SHARED((ACC_R, LAT), jnp.float32),
            pltpu.SemaphoreType.DMA,
            pltpu.SemaphoreType.DMA,
            pltpu.SemaphoreType.DMA,
            pltpu.SemaphoreType.DMA,
        ],
    )
    def sk(msg_hbm, seg_hbm, zero_hbm, out_hbm, seg_v, bufs, acc,
           lsem0, lsem1, lsem2, osem):
        c = lax.axis_index("c")
        s = lax.axis_index("s")
        wid = s * 2 + c
        lsems = (lsem0, lsem1, lsem2)
        base = wid * (nch * CH)
        zcopy = pltpu.make_async_copy(
            zero_hbm.at[pl.ds(0, 256)], acc.at[pl.ds(s * 256, 256)], osem)
        zcopy.start()
        pltpu.sync_copy(seg_hbm.at[pl.ds(wid * 24, 24)], seg_v)

        def lcopy(j, b):
            return pltpu.make_async_copy(
                msg_hbm.at[pl.ds(base + j * CH, CH)], bufs.at[b], lsems[b])

        for j in range(nbuf):
            lcopy(j, j).start()
        zcopy.wait()
        plsc.subcore_barrier()
        for j in range(nch):
            b = j % nbuf
            lcopy(j, b).wait()
            pltpu.sync_copy(bufs.at[b], acc.at[seg_v.at[j]], add=True)
            if j + nbuf < nch:
                lcopy(j + nbuf, b).start()
        plsc.subcore_barrier()
        pltpu.sync_copy(acc.at[pl.ds(s * 256, 128)], bufs.at[0])
        pltpu.make_async_copy(
            bufs.at[0], out_hbm.at[c, pl.ds(s * 256, 128)], osem).start()
        pltpu.sync_copy(acc.at[pl.ds(s * 256 + 128, 128)], bufs.at[1])
        pltpu.make_async_copy(
            bufs.at[1], out_hbm.at[c, pl.ds(s * 256 + 128, 128)], osem).start()
        pltpu.make_async_copy(
            bufs.at[0], out_hbm.at[c, pl.ds(s * 256, 128)], osem).wait()
        pltpu.make_async_copy(
            bufs.at[1], out_hbm.at[c, pl.ds(s * 256 + 128, 128)], osem).wait()

    _SC_CACHE["scatter"] = sk
    return sk(msgs, seg, zeros)


def _knn_body(rows_ref, cols_ref, tgt_ref):
    rp = rows_ref[0]
    cp = cols_ref[0]
    dm = jnp.zeros((NNB, NN), jnp.float32)
    for d in range(NPOS):
        diff = rp[:, d:d + 1] - cp[d:d + 1, :]
        dm = dm + diff * diff
    big = jnp.float32(1e10)
    dm = jnp.where(cp[3:4, :] > 0, dm, big)
    dm = jnp.where(rp[:, 3:4] > 0, dm, big)
    iota = lax.broadcasted_iota(jnp.int32, (NNB, NN), 1)
    inf = jnp.float32(np.inf)
    for t in range(KNN_K):
        vmin = jnp.min(dm, axis=1, keepdims=True)
        cand = jnp.where(dm == vmin, iota, jnp.int32(1 << 30))
        imin = jnp.min(cand, axis=1, keepdims=True)
        tgt_ref[0, :, t:t + 1] = imin
        dm = jnp.where(iota == imin, inf, dm)


def _knn(rows_p, cols_p):
    nb = NN // NNB
    return pl.pallas_call(
        _knn_body,
        grid=(NB, nb),
        in_specs=[
            pl.BlockSpec((1, NNB, 8), lambda g, r: (g, r, 0)),
            pl.BlockSpec((1, 8, NN), lambda g, r: (g, 0, 0)),
        ],
        out_specs=pl.BlockSpec((1, NNB, KNN_K), lambda g, r: (g, r, 0)),
        out_shape=jax.ShapeDtypeStruct((NB, NN, KNN_K), jnp.int32),
    )(rows_p, cols_p)


def _prep_body(cv_ref, w1, b1, w2, b2, w3, b3, wg, bg, we, be, wn, bn,
               ce_ref, cn_ref):
    li = lax.broadcasted_iota(jnp.int32, (8, LAT), 1)
    x = cv_ref[...]
    x = jnp.where(li < 16, jnp.sin(x), jnp.where(li < 32, jnp.cos(x), x))
    x = jnp.where(li < 34, x, jnp.float32(0.0))
    x = jax.nn.gelu(jnp.dot(x, w1[...], preferred_element_type=jnp.float32) + b1[0:1])
    x = jax.nn.gelu(jnp.dot(x, w2[...], preferred_element_type=jnp.float32) + b2[0:1])
    x = jnp.dot(x, w3[...], preferred_element_type=jnp.float32) + b3[0:1]
    g = jnp.dot(x, wg[...], preferred_element_type=jnp.float32) + bg[0:1]
    for s in range(MP_S):
        ce_ref[s] = jnp.dot(g, we[s], preferred_element_type=jnp.float32) + be[s, 0:1]
        cn_ref[s] = jnp.dot(g, wn[s], preferred_element_type=jnp.float32) + bn[s, 0:1]


def _prep(cv, w1, b1, w2, b2, w3, b3, wg, bg, we, be, wn, bn):
    spec0 = lambda shp: pl.BlockSpec(shp, lambda: tuple(0 for _ in shp))
    args = [cv, w1, b1, w2, b2, w3, b3, wg, bg, we, be, wn, bn]
    return pl.pallas_call(
        _prep_body,
        grid=(),
        in_specs=[spec0(a.shape) for a in args],
        out_specs=[spec0((MP_S, 8, LAT)), spec0((MP_S, 8, LAT))],
        out_shape=[jax.ShapeDtypeStruct((MP_S, 8, LAT), jnp.float32)] * 2,
    )(*args)


def _embed_body(z_ref, w_ref, b_ref, out_ref):
    out_ref[0] = (jnp.dot(z_ref[0], w_ref[...], preferred_element_type=jnp.float32)
                  + b_ref[0:1])


def _embed(zp, w, b):
    nb = NN // NNB
    return pl.pallas_call(
        _embed_body,
        grid=(NB, nb),
        in_specs=[
            pl.BlockSpec((1, NNB, 8), lambda g, r: (g, r, 0)),
            pl.BlockSpec((8, LAT), lambda g, r: (0, 0)),
            pl.BlockSpec((8, LAT), lambda g, r: (0, 0)),
        ],
        out_specs=pl.BlockSpec((1, NNB, LAT), lambda g, r: (g, r, 0)),
        out_shape=jax.ShapeDtypeStruct((NB, NN, LAT), jnp.float32),
    )(zp, w, b)


def _hab_body(h_ref, w_ref, bias_ref, out_ref):
    out_ref[0] = (jnp.dot(h_ref[0], w_ref[0], preferred_element_type=jnp.float32)
                  + bias_ref[0, 0, 0:1])


def _hab(h, wab, bias):
    return pl.pallas_call(
        _hab_body,
        grid=(NB, 2, 2),
        in_specs=[
            pl.BlockSpec((1, 1000, LAT), lambda g, j, r: (g, r, 0)),
            pl.BlockSpec((1, LAT, LAT), lambda g, j, r: (j, 0, 0)),
            pl.BlockSpec((1, 1, 8, LAT), lambda g, j, r: (g, j, 0, 0)),
        ],
        out_specs=pl.BlockSpec((1, 1000, LAT), lambda g, j, r: (2 * g + j, r, 0)),
        out_shape=jax.ShapeDtypeStruct((2 * NB, NNP, LAT), jnp.float32),
    )(h, wab, bias)


def _edge_body(gb_ref, ha_ref, w2, b2, w3, b3, w4, b4, out_ref):
    ha = ha_ref[0]
    ha = jnp.broadcast_to(ha[:, None, :], (EBLK, KNN_K, LAT))
    ha = ha.reshape(EBLK * KNN_K, LAT)
    x = gb_ref[...] + ha
    x = jax.nn.gelu(x)
    x = jnp.dot(x, w2[...], preferred_element_type=jnp.float32) + b2[0:1]
    x = jax.nn.gelu(x)
    x = jnp.dot(x, w3[...], preferred_element_type=jnp.float32) + b3[0:1]
    x = jax.nn.gelu(x)
    out_ref[...] = jnp.dot(x, w4[...], preferred_element_type=jnp.float32) + b4[0:1]


def _edge(gath_e, gath_s, w2, b2, w3, b3, w4, b4):
    ne = NNP // EBLK
    eb = EBLK * KNN_K
    wspec = pl.BlockSpec((LAT, LAT), lambda g, e: (0, 0))
    bspec = pl.BlockSpec((8, LAT), lambda g, e: (0, 0))
    return pl.pallas_call(
        _edge_body,
        grid=(NB, ne),
        in_specs=[
            pl.BlockSpec((eb, LAT), lambda g, e: (g * ne + e, 0)),
            pl.BlockSpec((1, EBLK, LAT), lambda g, e: (g, 2 * e, 0)),
            wspec, bspec, wspec, bspec, wspec, bspec,
        ],
        out_specs=pl.BlockSpec((eb, LAT), lambda g, e: (g * ne + e, 0)),
        out_shape=jax.ShapeDtypeStruct((ETOT, LAT), jnp.float32),
    )(gath_e, gath_s, w2, b2, w3, b3, w4, b4)


def _node_body(h_ref, p0_ref, p1_ref, w1a, w1b, cn_ref, w2, b2, w3, b3, w4, b4,
               out_ref):
    h = h_ref[0]
    agg = p0_ref[0] + p1_ref[0]
    x = (jnp.dot(h, w1a[...], preferred_element_type=jnp.float32)
         + jnp.dot(agg, w1b[...], preferred_element_type=jnp.float32)
         + cn_ref[0, 0:1])
    x = jax.nn.gelu(x)
    x = jnp.dot(x, w2[...], preferred_element_type=jnp.float32) + b2[0:1]
    x = jax.nn.gelu(x)
    x = jnp.dot(x, w3[...], preferred_element_type=jnp.float32) + b3[0:1]
    x = jax.nn.gelu(x)
    x = jnp.dot(x, w4[...], preferred_element_type=jnp.float32) + b4[0:1]
    out_ref[0] = h + x


def _node(h, parts, w1a, w1b, cn, w2, b2, w3, b3, w4, b4):
    nb = NN // NNB
    wspec = pl.BlockSpec((LAT, LAT), lambda g, r: (0, 0))
    bspec = pl.BlockSpec((8, LAT), lambda g, r: (0, 0))
    return pl.pallas_call(
        _node_body,
        grid=(NB, nb),
        in_specs=[
            pl.BlockSpec((1, NNB, LAT), lambda g, r: (g, r, 0)),
            pl.BlockSpec((1, NNB, LAT), lambda g, r: (0, g * nb + r, 0)),
            pl.BlockSpec((1, NNB, LAT), lambda g, r: (1, g * nb + r, 0)),
            wspec, wspec,
            pl.BlockSpec((1, 8, LAT), lambda g, r: (g, 0, 0)),
            wspec, bspec, wspec, bspec, wspec, bspec,
        ],
        out_specs=pl.BlockSpec((1, NNB, LAT), lambda g, r: (g, r, 0)),
        out_shape=jax.ShapeDtypeStruct((NB, NN, LAT), jnp.float32),
    )(h, parts, parts, w1a, w1b, cn, w2, b2, w3, b3, w4, b4)


def _dec_body(h_ref, zp_ref, w1, b1, w2, b2, w3, b3, w4, b4, out_ref):
    x = h_ref[0]
    x = jax.nn.gelu(jnp.dot(x, w1[...], preferred_element_type=jnp.float32) + b1[0:1])
    x = jax.nn.gelu(jnp.dot(x, w2[...], preferred_element_type=jnp.float32) + b2[0:1])
    x = jax.nn.gelu(jnp.dot(x, w3[...], preferred_element_type=jnp.float32) + b3[0:1])
    x = jnp.dot(x, w4[...], preferred_element_type=jnp.float32) + b4[0:1]
    out_ref[0] = zp_ref[0] - x[:, 0:8]


def _decode(h, zp, w1, b1, w2, b2, w3, b3, w4, b4):
    nb = NN // NNB
    wspec = pl.BlockSpec((LAT, LAT), lambda g, r: (0, 0))
    bspec = pl.BlockSpec((8, LAT), lambda g, r: (0, 0))
    return pl.pallas_call(
        _dec_body,
        grid=(NB, nb),
        in_specs=[
            pl.BlockSpec((1, NNB, LAT), lambda g, r: (g, r, 0)),
            pl.BlockSpec((1, NNB, 8), lambda g, r: (g, r, 0)),
            wspec, bspec, wspec, bspec, wspec, bspec, wspec, bspec,
        ],
        out_specs=pl.BlockSpec((1, NNB, 8), lambda g, r: (g, r, 0)),
        out_shape=jax.ShapeDtypeStruct((NB, NN, 8), jnp.float32),
    )(h, zp, w1, b1, w2, b2, w3, b3, w4, b4)


def _pad2(w, r, c):
    return jnp.pad(w, ((0, r - w.shape[0]), (0, c - w.shape[1])))


def _brow(b, c):
    return jnp.broadcast_to(jnp.pad(b, (0, c - b.shape[0]))[None, :], (8, c))


def kernel(z, t, conditioning, mask, params):
    f32 = jnp.float32
    z = z.astype(f32)

    half = 16
    freq = jnp.exp(jnp.arange(half, dtype=f32) * (-np.log(10000.0) / (half - 1)))
    phase = (t.astype(f32) * 1000.0)[:, None] * freq[None, :]
    cv = jnp.zeros((8, LAT), f32)
    cv = cv.at[0:NB, 0:half].set(phase)
    cv = cv.at[0:NB, half:2 * half].set(phase)
    cv = cv.at[0:NB, 2 * half:2 * half + 2].set(conditioning.astype(f32))

    (w1c, b1c), (w2c, b2c), (w3c, b3c) = params['cond_mlp']
    wg, bg = params['global_embed']
    we = jnp.stack([_pad2(params['edge_mlps'][s][0][0][256:384], LAT, LAT)
                    for s in range(MP_S)])
    be = jnp.stack([_brow(params['edge_mlps'][s][0][1], LAT) for s in range(MP_S)])
    wn = jnp.stack([_pad2(params['node_mlps'][s][0][0][256:384], LAT, LAT)
                    for s in range(MP_S)])
    bn = jnp.stack([_brow(params['node_mlps'][s][0][1], LAT) for s in range(MP_S)])
    cedge, cnode = _prep(
        cv, _pad2(w1c, LAT, 256), _brow(b1c, 256), _pad2(w2c, 256, 256),
        _brow(b2c, 256), _pad2(w3c, 256, LAT), _brow(b3c, LAT),
        _pad2(wg, LAT, LAT), _brow(bg, LAT), we, be, wn, bn)

    maskf = mask.astype(f32)
    pos = z[:, :, :NPOS]
    rows_p = jnp.concatenate(
        [pos, maskf[:, :, None], jnp.zeros((NB, NN, 4), f32)], axis=2)
    cols_p = jnp.swapaxes(rows_p, 1, 2)
    tgt = _knn(rows_p, cols_p)
    src0 = tgt[:, :, 0]

    gidx_e = jnp.pad(tgt, ((0, 0), (0, NNP - NN), (0, 0)))
    gidx_e = gidx_e.reshape(NB, NW, 10, CH)
    gidx_s = jnp.pad(src0, ((0, 0), (0, NNP - NN))).reshape(NB, NW, 64)
    gidx_s = jnp.pad(gidx_s, ((0, 0), (0, 0), (0, CH - 64)))[:, :, None, :]
    idxs = jnp.concatenate(
        [gidx_e, gidx_s, jnp.zeros((NB, NW, 5, CH), jnp.int32)],
        axis=2).reshape(NB, NW * 16, CH)
    seg = jnp.pad(tgt + (jnp.arange(NB, dtype=jnp.int32) * NN)[:, None, None],
                  ((0, 0), (0, NNP - NN), (0, 0)), constant_values=DUMP)
    seg = seg.reshape(NW, ETOT // (NW * CH), CH)
    seg = jnp.pad(seg, ((0, 0), (0, 24 - seg.shape[1]), (0, 0))).reshape(NW * 24, CH)
    zeros256 = jnp.zeros((256, LAT), f32)

    wemb, bemb = params['node_embed']
    zp = jnp.concatenate([z, jnp.zeros((NB, NN, 1), f32)], axis=2)
    h = _embed(zp, _pad2(wemb, 8, LAT), _brow(bemb, LAT))

    for s in range(MP_S):
        ew = params['edge_mlps'][s]
        nw_ = params['node_mlps'][s]
        wab = jnp.stack([ew[0][0][0:128], ew[0][0][128:256]])
        biasE = jnp.concatenate(
            [jnp.broadcast_to(cedge[s][0:NB][:, None, None, :], (NB, 1, 8, LAT)),
             jnp.zeros((NB, 1, 8, LAT), f32)], axis=1)
        tab = _hab(h, wab, biasE).reshape(NB, 2 * NNP, LAT)
        gath_e, gath_s = _sc_gather(tab, idxs)
        msgs = _edge(gath_e, gath_s, ew[1][0], _brow(ew[1][1], LAT), ew[2][0],
                     _brow(ew[2][1], LAT), ew[3][0], _brow(ew[3][1], LAT))
        parts = _sc_scatter_add(msgs, seg, zeros256)
        cn = jnp.broadcast_to(cnode[s][0:NB][:, None, :], (NB, 8, LAT))
        h = _node(h, parts, nw_[0][0][0:128], nw_[0][0][128:256], cn,
                  nw_[1][0], _brow(nw_[1][1], LAT), nw_[2][0], _brow(nw_[2][1], LAT),
                  nw_[3][0], _brow(nw_[3][1], LAT))

    dw = params['decoder']
    eps8 = _decode(h, zp, dw[0][0], _brow(dw[0][1], LAT), dw[1][0],
                   _brow(dw[1][1], LAT), dw[2][0], _brow(dw[2][1], LAT),
                   _pad2(dw[3][0], LAT, LAT), _brow(dw[3][1], LAT))
    return eps8[:, :, :7]


# speedup vs baseline: 9.8190x; 1.0394x over previous
import functools

import jax
import jax.numpy as jnp
import numpy as np
from jax import lax
from jax.experimental import pallas as pl
from jax.experimental.pallas import tpu as pltpu
from jax.experimental.pallas import tpu_sc as plsc

KNN_K = 20
NPOS = 3
LAT = 128
NB = 2
NN = 2000
NNP = 2048
EPG = NNP * KNN_K
ETOT = NB * EPG
GTOT = ETOT + NB * NNP
NW = 32
CH = 128
ACC_R = 4096
DUMP = 4000
MP_S = 4
NNB = 400
EBLK = 64


_SC_CACHE = {}


def _sc_gather(taba, tabb, idxs):
    nch = 10
    nset = 3
    if "gather" in _SC_CACHE:
        return _SC_CACHE["gather"](taba, tabb, idxs)
    mesh = plsc.VectorSubcoreMesh(core_axis_name="c", subcore_axis_name="s")

    @functools.partial(
        pl.kernel,
        out_type=(jax.ShapeDtypeStruct((ETOT, LAT), jnp.float32),
                  jax.ShapeDtypeStruct((NB, NW * CH, LAT), jnp.float32)),
        mesh=mesh,
        scratch_types=[
            pltpu.VMEM((16, CH), jnp.int32),
            pltpu.VMEM((2 * nset, CH, LAT), jnp.float32),
            pltpu.VMEM_SHARED((NNP, LAT), jnp.float32),
            pltpu.SemaphoreType.DMA,
            pltpu.SemaphoreType.DMA,
            pltpu.SemaphoreType.DMA,
            pltpu.SemaphoreType.DMA,
        ],
    )
    def gk(taba_hbm, tabb_hbm, idx_hbm, oute_hbm, outs_hbm, idx_v, bufs, tabsh,
           tsem, gsem0, gsem1, osem):
        wid = lax.axis_index("s") * 2 + lax.axis_index("c")
        s = lax.axis_index("s")
        gsems = (gsem0, gsem1)
        ngrp = (nch + nset - 1) // nset

        def stage(g, half):
            src = tabb_hbm if half else taba_hbm
            tcopy = pltpu.make_async_copy(
                src.at[g, pl.ds(s * (NNP // 16), NNP // 16)],
                tabsh.at[pl.ds(s * (NNP // 16), NNP // 16)], tsem)
            tcopy.start()
            return tcopy

        for g in range(NB):
            tcopy = stage(g, 1)
            pltpu.sync_copy(idx_hbm.at[g, pl.ds(wid * 16, 16)], idx_v)
            tcopy.wait()
            plsc.subcore_barrier()

            def gcopy(grp, setb, b):
                j = grp * nset + b
                return pltpu.make_async_copy(
                    tabsh.at[idx_v.at[j]], bufs.at[setb * nset + b], gsems[setb])

            def ocopy(grp, setb, b):
                j = grp * nset + b
                return pltpu.make_async_copy(
                    bufs.at[setb * nset + b],
                    oute_hbm.at[pl.ds(g * EPG + wid * 1280 + j * CH, CH)], osem)

            def fire(grp, setb):
                for b in range(nset):
                    if grp * nset + b < nch:
                        gcopy(grp, setb, b).start()

            def drain_writes(grp, setb):
                for b in range(nset):
                    if grp * nset + b < nch:
                        gcopy(grp, setb, b).wait()
                for b in range(nset):
                    if grp * nset + b < nch:
                        ocopy(grp, setb, b).start()
                for b in range(nset):
                    if grp * nset + b < nch:
                        ocopy(grp, setb, b).wait()

            fire(0, 0)
            for grp in range(ngrp):
                if grp + 1 < ngrp:
                    fire(grp + 1, (grp + 1) % 2)
                drain_writes(grp, grp % 2)
            plsc.subcore_barrier()
            stage(g, 0).wait()
            plsc.subcore_barrier()
            srcg = pltpu.make_async_copy(
                tabsh.at[idx_v.at[10]], bufs.at[0], gsem0)
            srcg.start()
            srcg.wait()
            pltpu.sync_copy(bufs.at[0], outs_hbm.at[g, pl.ds(wid * CH, CH)])
            plsc.subcore_barrier()

    _SC_CACHE["gather"] = gk
    return gk(taba, tabb, idxs)


def _sc_scatter_add(msgs, seg, zeros):
    nch = ETOT // (NW * CH)
    nbuf = 3
    if "scatter" in _SC_CACHE:
        return _SC_CACHE["scatter"](msgs, seg, zeros)
    mesh = plsc.VectorSubcoreMesh(core_axis_name="c", subcore_axis_name="s")

    @functools.partial(
        pl.kernel,
        out_type=jax.ShapeDtypeStruct((2, ACC_R, LAT), jnp.float32),
        mesh=mesh,
        scratch_types=[
            pltpu.VMEM((24, CH), jnp.int32),
            pltpu.VMEM((nbuf, CH, LAT), jnp.float32),
            pltpu.VMEM_SHARED((ACC_R, LAT), jnp.float32),
            pltpu.SemaphoreType.DMA,
            pltpu.SemaphoreType.DMA,
            pltpu.SemaphoreType.DMA,
            pltpu.SemaphoreType.DMA,
        ],
    )
    def sk(msg_hbm, seg_hbm, zero_hbm, out_hbm, seg_v, bufs, acc,
           lsem0, lsem1, lsem2, osem):
        c = lax.axis_index("c")
        s = lax.axis_index("s")
        wid = s * 2 + c
        lsems = (lsem0, lsem1, lsem2)
        base = wid * (nch * CH)
        zcopy = pltpu.make_async_copy(
            zero_hbm.at[pl.ds(0, 256)], acc.at[pl.ds(s * 256, 256)], osem)
        zcopy.start()
        pltpu.sync_copy(seg_hbm.at[pl.ds(wid * 24, 24)], seg_v)

        def lcopy(j, b):
            return pltpu.make_async_copy(
                msg_hbm.at[pl.ds(base + j * CH, CH)], bufs.at[b], lsems[b])

        for j in range(nbuf):
            lcopy(j, j).start()
        zcopy.wait()
        plsc.subcore_barrier()
        for j in range(nch):
            b = j % nbuf
            lcopy(j, b).wait()
            pltpu.sync_copy(bufs.at[b], acc.at[seg_v.at[j]], add=True)
            if j + nbuf < nch:
                lcopy(j + nbuf, b).start()
        plsc.subcore_barrier()
        pltpu.sync_copy(acc.at[pl.ds(s * 256, 128)], bufs.at[0])
        pltpu.make_async_copy(
            bufs.at[0], out_hbm.at[c, pl.ds(s * 256, 128)], osem).start()
        pltpu.sync_copy(acc.at[pl.ds(s * 256 + 128, 128)], bufs.at[1])
        pltpu.make_async_copy(
            bufs.at[1], out_hbm.at[c, pl.ds(s * 256 + 128, 128)], osem).start()
        pltpu.make_async_copy(
            bufs.at[0], out_hbm.at[c, pl.ds(s * 256, 128)], osem).wait()
        pltpu.make_async_copy(
            bufs.at[1], out_hbm.at[c, pl.ds(s * 256 + 128, 128)], osem).wait()

    _SC_CACHE["scatter"] = sk
    return sk(msgs, seg, zeros)


def _knn_body(rows_ref, cols_ref, tgt_ref):
    rp = rows_ref[0]
    cp = cols_ref[0]
    dm = jnp.zeros((NNB, NN), jnp.float32)
    for d in range(NPOS):
        diff = rp[:, d:d + 1] - cp[d:d + 1, :]
        dm = dm + diff * diff
    big = jnp.float32(1e10)
    dm = jnp.where(cp[3:4, :] > 0, dm, big)
    dm = jnp.where(rp[:, 3:4] > 0, dm, big)
    iota = lax.broadcasted_iota(jnp.int32, (NNB, NN), 1)
    inf = jnp.float32(np.inf)
    for t in range(KNN_K):
        vmin = jnp.min(dm, axis=1, keepdims=True)
        cand = jnp.where(dm == vmin, iota, jnp.int32(1 << 30))
        imin = jnp.min(cand, axis=1, keepdims=True)
        tgt_ref[0, :, t:t + 1] = imin
        dm = jnp.where(iota == imin, inf, dm)


def _knn(rows_p, cols_p):
    nb = NN // NNB
    return pl.pallas_call(
        _knn_body,
        grid=(NB, nb),
        in_specs=[
            pl.BlockSpec((1, NNB, 8), lambda g, r: (g, r, 0)),
            pl.BlockSpec((1, 8, NN), lambda g, r: (g, 0, 0)),
        ],
        out_specs=pl.BlockSpec((1, NNB, KNN_K), lambda g, r: (g, r, 0)),
        out_shape=jax.ShapeDtypeStruct((NB, NN, KNN_K), jnp.int32),
    )(rows_p, cols_p)


def _prep_body(cv_ref, w1, b1, w2, b2, w3, b3, wg, bg, we, be, wn, bn,
               ce_ref, cn_ref):
    li = lax.broadcasted_iota(jnp.int32, (8, LAT), 1)
    x = cv_ref[...]
    x = jnp.where(li < 16, jnp.sin(x), jnp.where(li < 32, jnp.cos(x), x))
    x = jnp.where(li < 34, x, jnp.float32(0.0))
    x = jax.nn.gelu(jnp.dot(x, w1[...], preferred_element_type=jnp.float32) + b1[0:1])
    x = jax.nn.gelu(jnp.dot(x, w2[...], preferred_element_type=jnp.float32) + b2[0:1])
    x = jnp.dot(x, w3[...], preferred_element_type=jnp.float32) + b3[0:1]
    g = jnp.dot(x, wg[...], preferred_element_type=jnp.float32) + bg[0:1]
    for s in range(MP_S):
        ce_ref[s] = jnp.dot(g, we[s], preferred_element_type=jnp.float32) + be[s, 0:1]
        cn_ref[s] = jnp.dot(g, wn[s], preferred_element_type=jnp.float32) + bn[s, 0:1]


def _prep(cv, w1, b1, w2, b2, w3, b3, wg, bg, we, be, wn, bn):
    spec0 = lambda shp: pl.BlockSpec(shp, lambda: tuple(0 for _ in shp))
    args = [cv, w1, b1, w2, b2, w3, b3, wg, bg, we, be, wn, bn]
    return pl.pallas_call(
        _prep_body,
        grid=(),
        in_specs=[spec0(a.shape) for a in args],
        out_specs=[spec0((MP_S, 8, LAT)), spec0((MP_S, 8, LAT))],
        out_shape=[jax.ShapeDtypeStruct((MP_S, 8, LAT), jnp.float32)] * 2,
    )(*args)


def _embed_body(z_ref, w_ref, b_ref, wa, ce_ref, wb, out_ref, ta_ref, tb_ref):
    h0 = (jnp.dot(z_ref[0], w_ref[...], preferred_element_type=jnp.float32)
          + b_ref[0:1])
    out_ref[0] = h0
    ta_ref[0] = (jnp.dot(h0, wa[...], preferred_element_type=jnp.float32)
                 + ce_ref[0, 0:1])
    tb_ref[0] = jnp.dot(h0, wb[...], preferred_element_type=jnp.float32)


def _embed(zp, w, b, wa, ce, wb):
    nb = NN // NNB
    wspec = pl.BlockSpec((LAT, LAT), lambda g, r: (0, 0))
    tspec = pl.BlockSpec((1, NNB, LAT), lambda g, r: (g, r, 0))
    return pl.pallas_call(
        _embed_body,
        grid=(NB, nb),
        in_specs=[
            pl.BlockSpec((1, NNB, 8), lambda g, r: (g, r, 0)),
            pl.BlockSpec((8, LAT), lambda g, r: (0, 0)),
            pl.BlockSpec((8, LAT), lambda g, r: (0, 0)),
            wspec,
            pl.BlockSpec((1, 8, LAT), lambda g, r: (g, 0, 0)),
            wspec,
        ],
        out_specs=[tspec, tspec, tspec],
        out_shape=[jax.ShapeDtypeStruct((NB, NN, LAT), jnp.float32),
                   jax.ShapeDtypeStruct((NB, NNP, LAT), jnp.float32),
                   jax.ShapeDtypeStruct((NB, NNP, LAT), jnp.float32)],
    )(zp, w, b, wa, ce, wb)


def _edge_body(gb_ref, ha_ref, w2, b2, w3, b3, w4, b4, out_ref):
    ha = ha_ref[0]
    ha = jnp.broadcast_to(ha[:, None, :], (EBLK, KNN_K, LAT))
    ha = ha.reshape(EBLK * KNN_K, LAT)
    x = gb_ref[...] + ha
    x = jax.nn.gelu(x)
    x = jnp.dot(x, w2[...], preferred_element_type=jnp.float32) + b2[0:1]
    x = jax.nn.gelu(x)
    x = jnp.dot(x, w3[...], preferred_element_type=jnp.float32) + b3[0:1]
    x = jax.nn.gelu(x)
    out_ref[...] = jnp.dot(x, w4[...], preferred_element_type=jnp.float32) + b4[0:1]


def _edge(gath_e, gath_s, w2, b2, w3, b3, w4, b4):
    ne = NNP // EBLK
    eb = EBLK * KNN_K
    wspec = pl.BlockSpec((LAT, LAT), lambda g, e: (0, 0))
    bspec = pl.BlockSpec((8, LAT), lambda g, e: (0, 0))
    return pl.pallas_call(
        _edge_body,
        grid=(NB, ne),
        in_specs=[
            pl.BlockSpec((eb, LAT), lambda g, e: (g * ne + e, 0)),
            pl.BlockSpec((1, EBLK, LAT), lambda g, e: (g, 2 * e, 0)),
            wspec, bspec, wspec, bspec, wspec, bspec,
        ],
        out_specs=pl.BlockSpec((eb, LAT), lambda g, e: (g * ne + e, 0)),
        out_shape=jax.ShapeDtypeStruct((ETOT, LAT), jnp.float32),
    )(gath_e, gath_s, w2, b2, w3, b3, w4, b4)


def _node_body(h_ref, p0_ref, p1_ref, w1a, w1b, cn_ref, w2, b2, w3, b3, w4, b4,
               out_ref):
    h = h_ref[0]
    agg = p0_ref[0] + p1_ref[0]
    x = (jnp.dot(h, w1a[...], preferred_element_type=jnp.float32)
         + jnp.dot(agg, w1b[...], preferred_element_type=jnp.float32)
         + cn_ref[0, 0:1])
    x = jax.nn.gelu(x)
    x = jnp.dot(x, w2[...], preferred_element_type=jnp.float32) + b2[0:1]
    x = jax.nn.gelu(x)
    x = jnp.dot(x, w3[...], preferred_element_type=jnp.float32) + b3[0:1]
    x = jax.nn.gelu(x)
    x = jnp.dot(x, w4[...], preferred_element_type=jnp.float32) + b4[0:1]
    out_ref[0] = h + x


def _node_tab_body(h_ref, p0_ref, p1_ref, w1a, w1b, cn_ref, w2, b2, w3, b3,
                   w4, b4, wa, ce_ref, wb, out_ref, ta_ref, tb_ref):
    h = h_ref[0]
    agg = p0_ref[0] + p1_ref[0]
    x = (jnp.dot(h, w1a[...], preferred_element_type=jnp.float32)
         + jnp.dot(agg, w1b[...], preferred_element_type=jnp.float32)
         + cn_ref[0, 0:1])
    x = jax.nn.gelu(x)
    x = jnp.dot(x, w2[...], preferred_element_type=jnp.float32) + b2[0:1]
    x = jax.nn.gelu(x)
    x = jnp.dot(x, w3[...], preferred_element_type=jnp.float32) + b3[0:1]
    x = jax.nn.gelu(x)
    x = jnp.dot(x, w4[...], preferred_element_type=jnp.float32) + b4[0:1]
    hn = h + x
    out_ref[0] = hn
    ta_ref[0] = (jnp.dot(hn, wa[...], preferred_element_type=jnp.float32)
                 + ce_ref[0, 0:1])
    tb_ref[0] = jnp.dot(hn, wb[...], preferred_element_type=jnp.float32)


def _node_tab(h, parts, w1a, w1b, cn, w2, b2, w3, b3, w4, b4, wa, ce, wb):
    nb = NN // NNB
    wspec = pl.BlockSpec((LAT, LAT), lambda g, r: (0, 0))
    bspec = pl.BlockSpec((8, LAT), lambda g, r: (0, 0))
    tspec = pl.BlockSpec((1, NNB, LAT), lambda g, r: (g, r, 0))
    return pl.pallas_call(
        _node_tab_body,
        grid=(NB, nb),
        in_specs=[
            tspec,
            pl.BlockSpec((1, NNB, LAT), lambda g, r: (0, g * nb + r, 0)),
            pl.BlockSpec((1, NNB, LAT), lambda g, r: (1, g * nb + r, 0)),
            wspec, wspec,
            pl.BlockSpec((1, 8, LAT), lambda g, r: (g, 0, 0)),
            wspec, bspec, wspec, bspec, wspec, bspec,
            wspec,
            pl.BlockSpec((1, 8, LAT), lambda g, r: (g, 0, 0)),
            wspec,
        ],
        out_specs=[tspec, tspec, tspec],
        out_shape=[jax.ShapeDtypeStruct((NB, NN, LAT), jnp.float32),
                   jax.ShapeDtypeStruct((NB, NNP, LAT), jnp.float32),
                   jax.ShapeDtypeStruct((NB, NNP, LAT), jnp.float32)],
    )(h, parts, parts, w1a, w1b, cn, w2, b2, w3, b3, w4, b4, wa, ce, wb)


def _node(h, parts, w1a, w1b, cn, w2, b2, w3, b3, w4, b4):
    nb = NN // NNB
    wspec = pl.BlockSpec((LAT, LAT), lambda g, r: (0, 0))
    bspec = pl.BlockSpec((8, LAT), lambda g, r: (0, 0))
    return pl.pallas_call(
        _node_body,
        grid=(NB, nb),
        in_specs=[
            pl.BlockSpec((1, NNB, LAT), lambda g, r: (g, r, 0)),
            pl.BlockSpec((1, NNB, LAT), lambda g, r: (0, g * nb + r, 0)),
            pl.BlockSpec((1, NNB, LAT), lambda g, r: (1, g * nb + r, 0)),
            wspec, wspec,
            pl.BlockSpec((1, 8, LAT), lambda g, r: (g, 0, 0)),
            wspec, bspec, wspec, bspec, wspec, bspec,
        ],
        out_specs=pl.BlockSpec((1, NNB, LAT), lambda g, r: (g, r, 0)),
        out_shape=jax.ShapeDtypeStruct((NB, NN, LAT), jnp.float32),
    )(h, parts, parts, w1a, w1b, cn, w2, b2, w3, b3, w4, b4)


def _dec_body(h_ref, zp_ref, w1, b1, w2, b2, w3, b3, w4, b4, out_ref):
    x = h_ref[0]
    x = jax.nn.gelu(jnp.dot(x, w1[...], preferred_element_type=jnp.float32) + b1[0:1])
    x = jax.nn.gelu(jnp.dot(x, w2[...], preferred_element_type=jnp.float32) + b2[0:1])
    x = jax.nn.gelu(jnp.dot(x, w3[...], preferred_element_type=jnp.float32) + b3[0:1])
    x = jnp.dot(x, w4[...], preferred_element_type=jnp.float32) + b4[0:1]
    out_ref[0] = zp_ref[0] - x[:, 0:8]


def _decode(h, zp, w1, b1, w2, b2, w3, b3, w4, b4):
    nb = NN // NNB
    wspec = pl.BlockSpec((LAT, LAT), lambda g, r: (0, 0))
    bspec = pl.BlockSpec((8, LAT), lambda g, r: (0, 0))
    return pl.pallas_call(
        _dec_body,
        grid=(NB, nb),
        in_specs=[
            pl.BlockSpec((1, NNB, LAT), lambda g, r: (g, r, 0)),
            pl.BlockSpec((1, NNB, 8), lambda g, r: (g, r, 0)),
            wspec, bspec, wspec, bspec, wspec, bspec, wspec, bspec,
        ],
        out_specs=pl.BlockSpec((1, NNB, 8), lambda g, r: (g, r, 0)),
        out_shape=jax.ShapeDtypeStruct((NB, NN, 8), jnp.float32),
    )(h, zp, w1, b1, w2, b2, w3, b3, w4, b4)


def _pad2(w, r, c):
    return jnp.pad(w, ((0, r - w.shape[0]), (0, c - w.shape[1])))


def _brow(b, c):
    return jnp.broadcast_to(jnp.pad(b, (0, c - b.shape[0]))[None, :], (8, c))


def kernel(z, t, conditioning, mask, params):
    f32 = jnp.float32
    z = z.astype(f32)

    half = 16
    freq = jnp.exp(jnp.arange(half, dtype=f32) * (-np.log(10000.0) / (half - 1)))
    phase = (t.astype(f32) * 1000.0)[:, None] * freq[None, :]
    cv = jnp.zeros((8, LAT), f32)
    cv = cv.at[0:NB, 0:half].set(phase)
    cv = cv.at[0:NB, half:2 * half].set(phase)
    cv = cv.at[0:NB, 2 * half:2 * half + 2].set(conditioning.astype(f32))

    (w1c, b1c), (w2c, b2c), (w3c, b3c) = params['cond_mlp']
    wg, bg = params['global_embed']
    we = jnp.stack([_pad2(params['edge_mlps'][s][0][0][256:384], LAT, LAT)
                    for s in range(MP_S)])
    be = jnp.stack([_brow(params['edge_mlps'][s][0][1], LAT) for s in range(MP_S)])
    wn = jnp.stack([_pad2(params['node_mlps'][s][0][0][256:384], LAT, LAT)
                    for s in range(MP_S)])
    bn = jnp.stack([_brow(params['node_mlps'][s][0][1], LAT) for s in range(MP_S)])
    cedge, cnode = _prep(
        cv, _pad2(w1c, LAT, 256), _brow(b1c, 256), _pad2(w2c, 256, 256),
        _brow(b2c, 256), _pad2(w3c, 256, LAT), _brow(b3c, LAT),
        _pad2(wg, LAT, LAT), _brow(bg, LAT), we, be, wn, bn)

    maskf = mask.astype(f32)
    pos = z[:, :, :NPOS]
    rows_p = jnp.concatenate(
        [pos, maskf[:, :, None], jnp.zeros((NB, NN, 4), f32)], axis=2)
    cols_p = jnp.swapaxes(rows_p, 1, 2)
    tgt = _knn(rows_p, cols_p)
    src0 = tgt[:, :, 0]

    gidx_e = jnp.pad(tgt, ((0, 0), (0, NNP - NN), (0, 0)))
    gidx_e = gidx_e.reshape(NB, NW, 10, CH)
    gidx_s = jnp.pad(src0, ((0, 0), (0, NNP - NN))).reshape(NB, NW, 64)
    gidx_s = jnp.pad(gidx_s, ((0, 0), (0, 0), (0, CH - 64)))[:, :, None, :]
    idxs = jnp.concatenate(
        [gidx_e, gidx_s, jnp.zeros((NB, NW, 5, CH), jnp.int32)],
        axis=2).reshape(NB, NW * 16, CH)
    seg = jnp.pad(tgt + (jnp.arange(NB, dtype=jnp.int32) * NN)[:, None, None],
                  ((0, 0), (0, NNP - NN), (0, 0)), constant_values=DUMP)
    seg = seg.reshape(NW, ETOT // (NW * CH), CH)
    seg = jnp.pad(seg, ((0, 0), (0, 24 - seg.shape[1]), (0, 0))).reshape(NW * 24, CH)
    zeros256 = jnp.zeros((256, LAT), f32)

    wemb, bemb = params['node_embed']
    zp = jnp.concatenate([z, jnp.zeros((NB, NN, 1), f32)], axis=2)
    ce = [jnp.broadcast_to(cedge[s][0:NB][:, None, :], (NB, 8, LAT))
          for s in range(MP_S)]
    ew0 = params['edge_mlps'][0]
    h, tabA, tabB = _embed(zp, _pad2(wemb, 8, LAT), _brow(bemb, LAT),
                           ew0[0][0][0:128], ce[0], ew0[0][0][128:256])

    for s in range(MP_S):
        ew = params['edge_mlps'][s]
        nw_ = params['node_mlps'][s]
        gath_e, gath_s = _sc_gather(tabA, tabB, idxs)
        msgs = _edge(gath_e, gath_s, ew[1][0], _brow(ew[1][1], LAT), ew[2][0],
                     _brow(ew[2][1], LAT), ew[3][0], _brow(ew[3][1], LAT))
        parts = _sc_scatter_add(msgs, seg, zeros256)
        cn = jnp.broadcast_to(cnode[s][0:NB][:, None, :], (NB, 8, LAT))
        nargs = (h, parts, nw_[0][0][0:128], nw_[0][0][128:256], cn,
                 nw_[1][0], _brow(nw_[1][1], LAT), nw_[2][0], _brow(nw_[2][1], LAT),
                 nw_[3][0], _brow(nw_[3][1], LAT))
        if s + 1 < MP_S:
            ewn = params['edge_mlps'][s + 1]
            h, tabA, tabB = _node_tab(
                *nargs, ewn[0][0][0:128], ce[s + 1], ewn[0][0][128:256])
        else:
            h = _node(*nargs)

    dw = params['decoder']
    eps8 = _decode(h, zp, dw[0][0], _brow(dw[0][1], LAT), dw[1][0],
                   _brow(dw[1][1], LAT), dw[2][0], _brow(dw[2][1], LAT),
                   _pad2(dw[3][0], LAT, LAT), _brow(dw[3][1], LAT))
    return eps8[:, :, :7]
